# baseline (device time: 6045 ns/iter reference)
import jax
import jax.numpy as jnp
from jax import lax
from jax.experimental import pallas as pl
from jax.experimental.pallas import tpu as pltpu

N_GLOBAL = 512


def kernel(x):
    m, n = x.shape
    rows, lanes = m // 128, 128

    def body(x_ref, out_ref, comm_ref, send_sem, recv_sem):
        my_x = lax.axis_index("x")
        my_y = lax.axis_index("y")
        peer = (my_x, 1 - my_y)

        barrier_sem = pltpu.get_barrier_semaphore()
        pl.semaphore_signal(
            barrier_sem, inc=1, device_id=peer,
            device_id_type=pl.DeviceIdType.MESH,
        )

        ones_n = jnp.ones((n, 1), dtype=jnp.float32)
        partial = jnp.dot(
            x_ref[:, :], ones_n, preferred_element_type=jnp.float32
        )
        comm_ref[0, :, :] = partial.reshape(rows, lanes)

        pl.semaphore_wait(barrier_sem, 1)

        rdma = pltpu.make_async_remote_copy(
            src_ref=comm_ref.at[0],
            dst_ref=comm_ref.at[1],
            send_sem=send_sem,
            recv_sem=recv_sem,
            device_id=peer,
            device_id_type=pl.DeviceIdType.MESH,
        )
        rdma.start()
        rdma.wait()

        total = comm_ref[0, :, :] + comm_ref[1, :, :]

        blk = lax.broadcasted_iota(jnp.int32, (m, rows), 0) // lanes
        i_id = lax.broadcasted_iota(jnp.int32, (m, rows), 1)
        onehot = (blk == i_id).astype(jnp.float32)
        rep = jnp.dot(
            onehot, total, preferred_element_type=jnp.float32
        )
        r_id = lax.broadcasted_iota(jnp.int32, (m, lanes), 0)
        l_id = lax.broadcasted_iota(jnp.int32, (m, lanes), 1)
        mask = (l_id == r_id % lanes).astype(jnp.float32)
        ones_l = jnp.ones((lanes, 1), dtype=jnp.float32)
        out_ref[:, :] = jnp.dot(
            rep * mask, ones_l, preferred_element_type=jnp.float32
        ) * (1.0 / N_GLOBAL)

    return pl.pallas_call(
        body,
        out_shape=jax.ShapeDtypeStruct((m, 1), jnp.float32),
        in_specs=[pl.BlockSpec(memory_space=pltpu.VMEM)],
        out_specs=pl.BlockSpec(memory_space=pltpu.VMEM),
        scratch_shapes=[
            pltpu.VMEM((2, rows, lanes), jnp.float32),
            pltpu.SemaphoreType.DMA,
            pltpu.SemaphoreType.DMA,
        ],
        compiler_params=pltpu.CompilerParams(collective_id=0),
    )(x)


# device time: 5883 ns/iter; 1.0275x vs baseline; 1.0275x over previous
import jax
import jax.numpy as jnp
from jax import lax
from jax.experimental import pallas as pl
from jax.experimental.pallas import tpu as pltpu

N_GLOBAL = 512


def kernel(x):
    m, n = x.shape
    rows, lanes = m // 128, 128

    def body(x_ref, out_ref, comm_ref, send_sem, recv_sem):
        my_x = lax.axis_index("x")
        my_y = lax.axis_index("y")
        peer = (my_x, 1 - my_y)

        barrier_sem = pltpu.get_barrier_semaphore()
        pl.semaphore_signal(
            barrier_sem, inc=1, device_id=peer,
            device_id_type=pl.DeviceIdType.MESH,
        )

        ones_n = jnp.full((n, 1), 1.0 / N_GLOBAL, dtype=jnp.float32)
        partial = jnp.dot(
            x_ref[:, :], ones_n, preferred_element_type=jnp.float32
        )
        comm_ref[0, :, :] = partial.reshape(rows, lanes)

        pl.semaphore_wait(barrier_sem, 1)

        rdma = pltpu.make_async_remote_copy(
            src_ref=comm_ref.at[0],
            dst_ref=comm_ref.at[1],
            send_sem=send_sem,
            recv_sem=recv_sem,
            device_id=peer,
            device_id_type=pl.DeviceIdType.MESH,
        )
        rdma.start()
        rdma.wait()

        total = comm_ref[0, :, :] + comm_ref[1, :, :]

        tot_t = total.T
        for i in range(rows):
            out_ref[pl.ds(i * lanes, lanes), :] = tot_t[:, i : i + 1]

    return pl.pallas_call(
        body,
        out_shape=jax.ShapeDtypeStruct((m, 1), jnp.float32),
        in_specs=[pl.BlockSpec(memory_space=pltpu.VMEM)],
        out_specs=pl.BlockSpec(memory_space=pltpu.VMEM),
        scratch_shapes=[
            pltpu.VMEM((2, rows, lanes), jnp.float32),
            pltpu.SemaphoreType.DMA,
            pltpu.SemaphoreType.DMA,
        ],
        compiler_params=pltpu.CompilerParams(collective_id=0),
    )(x)
